# baseline (speedup 1.0000x reference)
import functools

import jax
import jax.numpy as jnp
import numpy as np
from jax import lax
from jax.experimental import pallas as pl
from jax.experimental.pallas import tpu as pltpu
from jax.experimental.pallas import tpu_sc as plsc

_F32 = jnp.float32
_NC, _NS = 2, 16
_NW = _NC * _NS
_NPAD = 10240
_CH = 4000


def _sc_gather16(flat_idx, table):
    M = flat_idx.shape[0]
    per_w = M // _NW
    steps = per_w // _CH
    mesh = plsc.VectorSubcoreMesh(core_axis_name="c", subcore_axis_name="s")

    @functools.partial(
        pl.kernel, mesh=mesh,
        out_type=jax.ShapeDtypeStruct((M, 16), _F32),
        scratch_types=[
            pltpu.VMEM((_CH,), jnp.int32),
            pltpu.VMEM((_CH, 16), _F32),
            pltpu.SemaphoreType.DMA,
        ],
        compiler_params=pltpu.CompilerParams(use_tc_tiling_on_sc=False),
    )
    def k(idx_hbm, table_hbm, out_hbm, idx_v, rows_v, sem):
        wid = lax.axis_index("s") * _NC + lax.axis_index("c")
        base = wid * per_w

        def body(i, carry):
            off = base + i * _CH
            pltpu.sync_copy(idx_hbm.at[pl.ds(off, _CH)], idx_v)
            pltpu.async_copy(table_hbm.at[idx_v], rows_v, sem).wait()
            pltpu.sync_copy(rows_v, out_hbm.at[pl.ds(off, _CH)])
            return carry

        lax.fori_loop(0, steps, body, 0)

    return k(flat_idx, table)


def _sc_scatter_add16(payload, flat_idx, zeros_tbl):
    E = payload.shape[0]
    per_w = E // _NW
    steps = per_w // _CH
    rpt = _NPAD // _NS
    mesh = plsc.VectorSubcoreMesh(core_axis_name="c", subcore_axis_name="s")

    @functools.partial(
        pl.kernel, mesh=mesh,
        out_type=jax.ShapeDtypeStruct((_NC * _NPAD, 16), _F32),
        scratch_types=[
            pltpu.VMEM((_CH,), jnp.int32),
            pltpu.VMEM((_CH, 16), _F32),
            pltpu.VMEM_SHARED((_NPAD, 16), _F32),
        ],
        compiler_params=pltpu.CompilerParams(use_tc_tiling_on_sc=False),
    )
    def k(pay_hbm, idx_hbm, zeros_hbm, out_hbm, idx_v, rows_v, acc_sh):
        cid = lax.axis_index("c")
        sid = lax.axis_index("s")
        wid = sid * _NC + cid
        r0 = sid * rpt
        pltpu.sync_copy(zeros_hbm.at[pl.ds(r0, rpt)], acc_sh.at[pl.ds(r0, rpt)])
        plsc.subcore_barrier()
        base = wid * per_w

        def body(i, carry):
            off = base + i * _CH
            pltpu.sync_copy(idx_hbm.at[pl.ds(E + off, _CH)], idx_v)
            pltpu.sync_copy(pay_hbm.at[pl.ds(off, _CH)], rows_v)
            pltpu.sync_copy(rows_v, acc_sh.at[idx_v], add=True)
            return carry

        lax.fori_loop(0, steps, body, 0)
        plsc.subcore_barrier()
        pltpu.sync_copy(acc_sh.at[pl.ds(r0, rpt)],
                        out_hbm.at[pl.ds(cid * _NPAD + r0, rpt)])

    return k(payload, flat_idx, zeros_tbl)


def _tc_node0(fc_xp, sc_xp, w_fc, w_sc, S0, S1):
    n = fc_xp.shape[0]

    def body(fx, sx, wf, ws, s0, s1, out):
        h_fc = jnp.maximum(fx[...] @ wf[...], 0.0)
        h_sc = jnp.maximum(sx[...] @ ws[...], 0.0)
        out[...] = h_fc @ s0[...] + h_sc @ s1[...]

    return pl.pallas_call(
        body,
        out_shape=jax.ShapeDtypeStruct((n, 16), _F32),
    )(fc_xp, sc_xp, w_fc, w_sc, S0, S1)


def _tc_edgeA(Gp, fc_ea, sc_ea, w, blk):
    R = Gp.shape[0] // 2
    grid = R // blk

    def body(g0, g1, ea_f, ea_s, sfc, wsck, k1f, u8, v8, febf, ew1f,
             k1s, fes, febs, ew1s, c0, c1, p2, p4, p12, out):
        relu = lambda x: jnp.maximum(x, 0.0)
        f3 = ea_f[...].reshape(blk, 8, 1)
        s3 = ea_s[...].reshape(blk, 8, 3)
        t32 = jnp.concatenate(
            [jnp.broadcast_to(f3, (blk, 8, 8)),
             jnp.broadcast_to(s3[:, :, 0:1], (blk, 8, 8)),
             jnp.broadcast_to(s3[:, :, 1:2], (blk, 8, 8)),
             jnp.broadcast_to(s3[:, :, 2:3], (blk, 8, 8))], axis=2)
        ey = jnp.eye(8, dtype=jnp.float32)
        msk = jnp.concatenate([ey, ey, ey, ey], axis=1)[None]
        x32 = jnp.sum(t32 * msk, axis=1)
        a_col = x32 @ sfc[...]
        ap = relu(a_col)
        am = relu(-a_col)
        hsum = g0[...] + g1[...]
        e2f = relu(hsum @ k1f[...] + ap @ u8[...] + am @ v8[...] + febf[...])
        e1bf = relu(e2f @ ew1f[...])
        e1as = relu(x32 @ wsck[...])
        e2s = relu(hsum @ k1s[...] + e1as @ fes[...] + febs[...])
        e1bs = relu(e2s @ ew1s[...])
        out[...] = (ap @ c0[...] + am @ c1[...] + e1bf @ p2[...]
                    + e1as @ p4[...] + e1bs @ p12[...])

    full = lambda shape: pl.BlockSpec(shape, lambda i: (0, 0))
    return pl.pallas_call(
        body,
        grid=(grid,),
        in_specs=[
            pl.BlockSpec((blk, 128), lambda i: (i, 0)),
            pl.BlockSpec((blk, 128), lambda i: (i + grid, 0)),
            pl.BlockSpec((blk * 8, 1), lambda i: (i, 0)),
            pl.BlockSpec((blk * 8, 3), lambda i: (i, 0)),
            full((32, 8)), full((32, 64)),
            full((128, 128)), full((8, 128)), full((8, 128)), full((1, 128)),
            full((128, 16)),
            full((128, 128)), full((64, 128)),
            full((1, 128)), full((128, 16)), full((8, 128)), full((8, 128)),
            full((16, 128)), full((64, 128)), full((16, 128)),
        ],
        out_specs=pl.BlockSpec((blk, 128), lambda i: (i, 0)),
        out_shape=jax.ShapeDtypeStruct((R, 128), _F32),
        compiler_params=pltpu.CompilerParams(
            dimension_semantics=("arbitrary",)),
    )(Gp, Gp, fc_ea, sc_ea, w["S_fc"], w["W_sce_k"],
      w["K1_fc_k"], w["U8"], w["V8"], w["feb_fc_t"],
      w["ew1_fc_k"],
      w["K1_sc_k"], w["few_e_sc_k"],
      w["feb_sc_t"], w["ew1_sc_k"], w["C0_k"], w["C1_k"], w["P2_k"],
      w["P4_k"], w["P12_k"])


def _tc_node1(partials, hc, w):
    n = _NPAD

    def body(p0, p1, h, m1f, m2f, fnbf, g0f, b0f, nw1f, m3f, fw1bf, fn1bf,
             g1f, b1f, m1s, m2s, fnbs, g0s, b0s, nw1s, m3s, fw1bs, fn1bs,
             g1s, b1s, q0, q1, out):
        relu = lambda x: jnp.maximum(x, 0.0)
        aggr = p0[...] + p1[...]
        hc_ = h[...]
        x1f = relu(aggr @ m1f[...] + hc_ @ m2f[...] + fnbf[...]) * g0f[...] + b0f[...]
        h1f = relu(x1f @ nw1f[...])
        zf = relu(aggr @ m3f[...] + h1f @ fw1bf[...] + fn1bf[...]) * g1f[...] + b1f[...]
        x1s = relu(aggr @ m1s[...] + hc_ @ m2s[...] + fnbs[...]) * g0s[...] + b0s[...]
        h1s = relu(x1s @ nw1s[...])
        zs = relu(aggr @ m3s[...] + h1s @ fw1bs[...] + fn1bs[...]) * g1s[...] + b1s[...]
        out[...] = zf @ q0[...] + zs @ q1[...]

    full = lambda shape: pl.BlockSpec(shape, lambda i: tuple(0 for _ in shape))
    return pl.pallas_call(
        body,
        grid=(1,),
        in_specs=[
            pl.BlockSpec((n, 16), lambda i: (0, 0)),
            pl.BlockSpec((n, 16), lambda i: (1, 0)),
            pl.BlockSpec((n, 16), lambda i: (0, 0)),
            full((16, 16)), full((16, 16)), full((1, 16)), full((1, 16)),
            full((1, 16)), full((16, 2)), full((16, 4)), full((2, 4)),
            full((1, 4)), full((1, 4)), full((1, 4)),
            full((16, 16)), full((16, 16)), full((1, 16)), full((1, 16)),
            full((1, 16)), full((16, 2)), full((16, 4)), full((2, 4)),
            full((1, 4)), full((1, 4)), full((1, 4)),
            full((4, 16)), full((4, 16)),
        ],
        out_specs=pl.BlockSpec((n, 16), lambda i: (0, 0)),
        out_shape=jax.ShapeDtypeStruct((n, 16), _F32),
    )(partials, partials, hc,
      w["M1_fc"], w["M2_fc"], w["fnb_fc"], w["g0_fc"], w["b0_fc"],
      w["nw1_fc"], w["M3_fc"], w["fn1w_b_fc"], w["fn1b_fc"], w["g1_fc"],
      w["b1_fc"],
      w["M1_sc"], w["M2_sc"], w["fnb_sc"], w["g0_sc"], w["b0_sc"],
      w["nw1_sc"], w["M3_sc"], w["fn1w_b_sc"], w["fn1b_sc"], w["g1_sc"],
      w["b1_sc"], w["Q0"], w["Q1"])


def _tc_decoder(Zp, w, blk):
    R2 = Zp.shape[0]
    R = R2 // 2
    grid = R // blk

    def body(z0, z1, w1, b1, w2, b2, w3, b3, w4, b4, out):
        relu = lambda x: jnp.maximum(x, 0.0)
        h = z0[...] * z1[...]
        l1 = relu(h @ w1[...] + b1[...])
        l2 = relu(l1 @ w2[...] + b2[...])
        l3 = relu(l2 @ w3[...] + b3[...])
        l4 = l3 @ w4[...] + b4[...]
        out[...] = 1.0 / (1.0 + jnp.exp(-l4))

    full = lambda shape: pl.BlockSpec(shape, lambda i: (0, 0))
    return pl.pallas_call(
        body,
        grid=(grid,),
        in_specs=[
            pl.BlockSpec((blk, 128), lambda i: (i, 0)),
            pl.BlockSpec((blk, 128), lambda i: (i + grid, 0)),
            full((128, 512)), full((1, 512)), full((512, 1024)),
            full((1, 1024)), full((1024, 256)), full((1, 256)),
            full((256, 8)), full((1, 8)),
        ],
        out_specs=pl.BlockSpec((blk, 8), lambda i: (i, 0)),
        out_shape=jax.ShapeDtypeStruct((R, 8), _F32),
        compiler_params=pltpu.CompilerParams(
            dimension_semantics=("arbitrary",)),
    )(Zp, Zp, w["W1k"], w["db1_t"], w["W2k"], w["db2_t"], w["W3k"],
      w["db3_t"], w["W4k"], w["db4_t"])


def _fold_weights(p):
    z8 = lambda *s: jnp.zeros(s, _F32)
    bn_s = np.float32(1.0 / np.sqrt(1.0 + 1e-5))
    eye16 = jnp.eye(16, dtype=_F32)
    w = {}
    I8 = jnp.eye(8, dtype=_F32)
    kr = lambda m: jnp.kron(I8, m)
    w_fce = p["fc0_edge_w"][0]
    few_fc = p["fc0_fe_w"]
    u_fc = (jnp.maximum(w_fce, 0) @ few_fc[8:])[None]
    v_fc = (jnp.maximum(-w_fce, 0) @ few_fc[8:])[None]
    w["U8"] = kr(u_fc)
    w["V8"] = kr(v_fc)
    w["K1_fc_k"] = kr(jnp.concatenate([few_fc[:8], z8(8, 16)], 0))
    w["feb_fc_t"] = jnp.tile(p["fc0_fe_b"][None], (1, 8))
    w["ew1_fc_k"] = kr(p["fc1_edge_w"])
    few_sc = p["sc0_fe_w"]
    w["K1_sc_k"] = kr(jnp.concatenate([z8(8, 16), few_sc[:8]], 0))
    w["few_e_sc_k"] = kr(few_sc[8:])
    w["feb_sc_t"] = jnp.tile(p["sc0_fe_b"][None], (1, 8))
    w["ew1_sc_k"] = kr(p["sc1_edge_w"])
    w["S_fc"] = jnp.concatenate([I8, jnp.zeros((24, 8), _F32)], 0)
    w["W_sce_k"] = jnp.concatenate(
        [jnp.zeros((8, 64), _F32)]
        + [kr(p["sc0_edge_w"][c:c + 1]) for c in range(3)], 0)
    w["C0_k"] = kr(eye16[0:1])
    w["C1_k"] = kr(eye16[1:2])
    w["P2_k"] = kr(eye16[2:4])
    w["P4_k"] = kr(eye16[4:12])
    w["P12_k"] = kr(eye16[12:14])
    Wrec_fc = z8(16, 8).at[0].set(jnp.maximum(w_fce, 0)).at[1].set(
        jnp.maximum(-w_fce, 0))
    w["M1_fc"] = Wrec_fc @ p["fc0_fn_w"][:8]
    w["M2_fc"] = jnp.concatenate([p["fc0_fn_w"][8:], z8(8, 16)], 0)
    w["fnb_fc"] = p["fc0_fn_b"][None]
    w["g0_fc"] = (p["fc_bn0_g"] * bn_s)[None]
    w["b0_fc"] = p["fc_bn0_b"][None]
    w["nw1_fc"] = p["fc1_node_w"]
    R24 = z8(16, 2).at[2, 0].set(1.0).at[3, 1].set(1.0)
    w["M3_fc"] = R24 @ p["fc1_fn_w"][:2]
    w["fn1w_b_fc"] = p["fc1_fn_w"][2:]
    w["fn1b_fc"] = p["fc1_fn_b"][None]
    w["g1_fc"] = (p["fc_bn1_g"] * bn_s)[None]
    w["b1_fc"] = p["fc_bn1_b"][None]
    R4_12 = z8(16, 8).at[4:12].set(jnp.eye(8, dtype=_F32))
    w["M1_sc"] = R4_12 @ p["sc0_fn_w"][:8]
    w["M2_sc"] = jnp.concatenate([z8(8, 16), p["sc0_fn_w"][8:]], 0)
    w["fnb_sc"] = p["sc0_fn_b"][None]
    w["g0_sc"] = (p["sc_bn0_g"] * bn_s)[None]
    w["b0_sc"] = p["sc_bn0_b"][None]
    w["nw1_sc"] = p["sc1_node_w"]
    R12_14 = z8(16, 2).at[12, 0].set(1.0).at[13, 1].set(1.0)
    w["M3_sc"] = R12_14 @ p["sc1_fn_w"][:2]
    w["fn1w_b_sc"] = p["sc1_fn_w"][2:]
    w["fn1b_sc"] = p["sc1_fn_b"][None]
    w["g1_sc"] = (p["sc_bn1_g"] * bn_s)[None]
    w["b1_sc"] = p["sc_bn1_b"][None]
    w["Q0"] = jnp.concatenate([jnp.eye(4, dtype=_F32), z8(4, 12)], 1)
    w["Q1"] = jnp.concatenate([z8(4, 4), jnp.eye(4, dtype=_F32), z8(4, 8)], 1)
    w["S0"] = jnp.concatenate([jnp.eye(8, dtype=_F32), z8(8, 8)], 1)
    w["S1"] = jnp.concatenate([z8(8, 8), jnp.eye(8, dtype=_F32)], 1)
    w["W1k"] = kr(jnp.concatenate([p["dec_w1"], z8(8, 64)], 0))
    w["db1_t"] = jnp.tile(p["dec_b1"][None], (1, 8))
    w["W2k"] = kr(p["dec_w2"])
    w["db2_t"] = jnp.tile(p["dec_b2"][None], (1, 8))
    w["W3k"] = kr(p["dec_w3"])
    w["db3_t"] = jnp.tile(p["dec_b3"][None], (1, 8))
    w["W4k"] = kr(p["dec_w4"])
    w["db4_t"] = jnp.tile(p["dec_b4"][None], (1, 8))
    return w


def kernel(fc_x, sc_x, fc_edge_attr, sc_edge_attr, edge_index, params):
    n = fc_x.shape[0]
    w = _fold_weights(params)
    fc_xp = jnp.pad(fc_x, ((0, _NPAD - n), (0, 0)))
    sc_xp = jnp.pad(sc_x, ((0, _NPAD - n), (0, 0)))
    flat_idx = edge_index.reshape(-1)
    zeros_tbl = jnp.zeros((_NPAD, 16), _F32)

    e = edge_index.shape[1]
    hc = _tc_node0(fc_xp, sc_xp, params["fc0_node_w"], params["sc0_node_w"],
                   w["S0"], w["S1"])
    G = _sc_gather16(flat_idx, hc)
    payload_p = _tc_edgeA(G.reshape(2 * e // 8, 128), fc_edge_attr,
                          sc_edge_attr, w, blk=1000)
    partials = _sc_scatter_add16(payload_p.reshape(e, 16), flat_idx, zeros_tbl)
    zpad = _tc_node1(partials, hc, w)
    Z = _sc_gather16(flat_idx, zpad)
    out_p = _tc_decoder(Z.reshape(2 * e // 8, 128), w, blk=2000)
    return out_p.reshape(e, 1)


# speedup vs baseline: 1.0101x; 1.0101x over previous
import functools

import jax
import jax.numpy as jnp
import numpy as np
from jax import lax
from jax.experimental import pallas as pl
from jax.experimental.pallas import tpu as pltpu
from jax.experimental.pallas import tpu_sc as plsc

_F32 = jnp.float32
_NC, _NS = 2, 16
_NW = _NC * _NS
_NPAD = 10240
_CH = 4000


def _sc_gather16(flat_idx, table):
    M = flat_idx.shape[0]
    per_w = M // _NW
    steps = per_w // _CH
    mesh = plsc.VectorSubcoreMesh(core_axis_name="c", subcore_axis_name="s")

    @functools.partial(
        pl.kernel, mesh=mesh,
        out_type=jax.ShapeDtypeStruct((M, 16), _F32),
        scratch_types=[
            pltpu.VMEM((_CH,), jnp.int32),
            pltpu.VMEM((_CH, 16), _F32),
            pltpu.SemaphoreType.DMA,
        ],
        compiler_params=pltpu.CompilerParams(use_tc_tiling_on_sc=False),
    )
    def k(idx_hbm, table_hbm, out_hbm, idx_v, rows_v, sem):
        wid = lax.axis_index("s") * _NC + lax.axis_index("c")
        base = wid * per_w

        def body(i, carry):
            off = base + i * _CH
            pltpu.sync_copy(idx_hbm.at[pl.ds(off, _CH)], idx_v)
            pltpu.async_copy(table_hbm.at[idx_v], rows_v, sem).wait()
            pltpu.sync_copy(rows_v, out_hbm.at[pl.ds(off, _CH)])
            return carry

        lax.fori_loop(0, steps, body, 0)

    return k(flat_idx, table)


def _sc_scatter_add16(payload, flat_idx, zeros_tbl):
    E = payload.shape[0]
    per_w = E // _NW
    steps = per_w // _CH
    rpt = _NPAD // _NS
    mesh = plsc.VectorSubcoreMesh(core_axis_name="c", subcore_axis_name="s")

    @functools.partial(
        pl.kernel, mesh=mesh,
        out_type=jax.ShapeDtypeStruct((_NC * _NPAD, 16), _F32),
        scratch_types=[
            pltpu.VMEM((_CH,), jnp.int32),
            pltpu.VMEM((_CH, 16), _F32),
            pltpu.VMEM_SHARED((_NPAD, 16), _F32),
        ],
        compiler_params=pltpu.CompilerParams(use_tc_tiling_on_sc=False),
    )
    def k(pay_hbm, idx_hbm, zeros_hbm, out_hbm, idx_v, rows_v, acc_sh):
        cid = lax.axis_index("c")
        sid = lax.axis_index("s")
        wid = sid * _NC + cid
        r0 = sid * rpt
        pltpu.sync_copy(zeros_hbm.at[pl.ds(r0, rpt)], acc_sh.at[pl.ds(r0, rpt)])
        plsc.subcore_barrier()
        base = wid * per_w

        def body(i, carry):
            off = base + i * _CH
            pltpu.sync_copy(idx_hbm.at[pl.ds(E + off, _CH)], idx_v)
            pltpu.sync_copy(pay_hbm.at[pl.ds(off, _CH)], rows_v)
            pltpu.sync_copy(rows_v, acc_sh.at[idx_v], add=True)
            return carry

        lax.fori_loop(0, steps, body, 0)
        plsc.subcore_barrier()
        pltpu.sync_copy(acc_sh.at[pl.ds(r0, rpt)],
                        out_hbm.at[pl.ds(cid * _NPAD + r0, rpt)])

    return k(payload, flat_idx, zeros_tbl)


def _tc_node0(fc_xp, sc_xp, w_fc, w_sc, S0, S1):
    n = fc_xp.shape[0]

    def body(fx, sx, wf, ws, s0, s1, out):
        h_fc = jnp.maximum(fx[...] @ wf[...], 0.0)
        h_sc = jnp.maximum(sx[...] @ ws[...], 0.0)
        out[...] = h_fc @ s0[...] + h_sc @ s1[...]

    return pl.pallas_call(
        body,
        out_shape=jax.ShapeDtypeStruct((n, 16), _F32),
    )(fc_xp, sc_xp, w_fc, w_sc, S0, S1)


def _tc_edgeA(Gp, fc_ea, sc_ea, w, blk):
    R = Gp.shape[0] // 2
    grid = R // blk

    def body(g0, g1, ea_f, ea_s, sfc, wsck, k1f, u8, v8, febf, ew1f,
             k1s, fes, febs, ew1s, c0, c1, p2, p4, p12, out):
        relu = lambda x: jnp.maximum(x, 0.0)
        f3 = ea_f[...].reshape(blk, 8, 1)
        s3 = ea_s[...].reshape(blk, 8, 3)
        t32 = jnp.concatenate(
            [jnp.broadcast_to(f3, (blk, 8, 8)),
             jnp.broadcast_to(s3[:, :, 0:1], (blk, 8, 8)),
             jnp.broadcast_to(s3[:, :, 1:2], (blk, 8, 8)),
             jnp.broadcast_to(s3[:, :, 2:3], (blk, 8, 8))], axis=2)
        ey = jnp.eye(8, dtype=jnp.float32)
        msk = jnp.concatenate([ey, ey, ey, ey], axis=1)[None]
        x32 = jnp.sum(t32 * msk, axis=1)
        a_col = x32 @ sfc[...]
        ap = relu(a_col)
        am = relu(-a_col)
        hsum = g0[...] + g1[...]
        e2f = relu(hsum @ k1f[...] + ap @ u8[...] + am @ v8[...] + febf[...])
        e1bf = relu(e2f @ ew1f[...])
        e1as = relu(x32 @ wsck[...])
        e2s = relu(hsum @ k1s[...] + e1as @ fes[...] + febs[...])
        e1bs = relu(e2s @ ew1s[...])
        out[...] = (ap @ c0[...] + am @ c1[...] + e1bf @ p2[...]
                    + e1as @ p4[...] + e1bs @ p12[...])

    full = lambda shape: pl.BlockSpec(shape, lambda i: (0, 0))
    return pl.pallas_call(
        body,
        grid=(grid,),
        in_specs=[
            pl.BlockSpec((blk, 128), lambda i: (i, 0)),
            pl.BlockSpec((blk, 128), lambda i: (i + grid, 0)),
            pl.BlockSpec((blk * 8, 1), lambda i: (i, 0)),
            pl.BlockSpec((blk * 8, 3), lambda i: (i, 0)),
            full((32, 8)), full((32, 64)),
            full((128, 128)), full((8, 128)), full((8, 128)), full((1, 128)),
            full((128, 16)),
            full((128, 128)), full((64, 128)),
            full((1, 128)), full((128, 16)), full((8, 128)), full((8, 128)),
            full((16, 128)), full((64, 128)), full((16, 128)),
        ],
        out_specs=pl.BlockSpec((blk, 128), lambda i: (i, 0)),
        out_shape=jax.ShapeDtypeStruct((R, 128), _F32),
        compiler_params=pltpu.CompilerParams(
            dimension_semantics=("arbitrary",)),
    )(Gp, Gp, fc_ea, sc_ea, w["S_fc"], w["W_sce_k"],
      w["K1_fc_k"], w["U8"], w["V8"], w["feb_fc_t"],
      w["ew1_fc_k"],
      w["K1_sc_k"], w["few_e_sc_k"],
      w["feb_sc_t"], w["ew1_sc_k"], w["C0_k"], w["C1_k"], w["P2_k"],
      w["P4_k"], w["P12_k"])


def _tc_node1(partials, hc, w):
    n = _NPAD

    def body(p0, p1, h, m1f, m2f, fnbf, g0f, b0f, nw1f, m3f, fw1bf, fn1bf,
             g1f, b1f, m1s, m2s, fnbs, g0s, b0s, nw1s, m3s, fw1bs, fn1bs,
             g1s, b1s, q0, q1, out):
        relu = lambda x: jnp.maximum(x, 0.0)
        aggr = p0[...] + p1[...]
        hc_ = h[...]
        x1f = relu(aggr @ m1f[...] + hc_ @ m2f[...] + fnbf[...]) * g0f[...] + b0f[...]
        h1f = relu(x1f @ nw1f[...])
        zf = relu(aggr @ m3f[...] + h1f @ fw1bf[...] + fn1bf[...]) * g1f[...] + b1f[...]
        x1s = relu(aggr @ m1s[...] + hc_ @ m2s[...] + fnbs[...]) * g0s[...] + b0s[...]
        h1s = relu(x1s @ nw1s[...])
        zs = relu(aggr @ m3s[...] + h1s @ fw1bs[...] + fn1bs[...]) * g1s[...] + b1s[...]
        out[...] = zf @ q0[...] + zs @ q1[...]

    full = lambda shape: pl.BlockSpec(shape, lambda i: tuple(0 for _ in shape))
    return pl.pallas_call(
        body,
        grid=(1,),
        in_specs=[
            pl.BlockSpec((n, 16), lambda i: (0, 0)),
            pl.BlockSpec((n, 16), lambda i: (1, 0)),
            pl.BlockSpec((n, 16), lambda i: (0, 0)),
            full((16, 16)), full((16, 16)), full((1, 16)), full((1, 16)),
            full((1, 16)), full((16, 2)), full((16, 4)), full((2, 4)),
            full((1, 4)), full((1, 4)), full((1, 4)),
            full((16, 16)), full((16, 16)), full((1, 16)), full((1, 16)),
            full((1, 16)), full((16, 2)), full((16, 4)), full((2, 4)),
            full((1, 4)), full((1, 4)), full((1, 4)),
            full((4, 16)), full((4, 16)),
        ],
        out_specs=pl.BlockSpec((n, 16), lambda i: (0, 0)),
        out_shape=jax.ShapeDtypeStruct((n, 16), _F32),
    )(partials, partials, hc,
      w["M1_fc"], w["M2_fc"], w["fnb_fc"], w["g0_fc"], w["b0_fc"],
      w["nw1_fc"], w["M3_fc"], w["fn1w_b_fc"], w["fn1b_fc"], w["g1_fc"],
      w["b1_fc"],
      w["M1_sc"], w["M2_sc"], w["fnb_sc"], w["g0_sc"], w["b0_sc"],
      w["nw1_sc"], w["M3_sc"], w["fn1w_b_sc"], w["fn1b_sc"], w["g1_sc"],
      w["b1_sc"], w["Q0"], w["Q1"])


def _tc_decoder(Zp, w, blk):
    R2 = Zp.shape[0]
    R = R2 // 2
    grid = R // blk

    def body(z0, z1, w1, b1, w2, b2, w3, b3, w4, b4, out):
        relu = lambda x: jnp.maximum(x, 0.0)
        h = z0[...] * z1[...]
        l1 = relu(h @ w1[...] + b1[...])
        l2 = relu(l1 @ w2[...] + b2[...])
        l3 = relu(l2 @ w3[...] + b3[...])
        l4 = l3 @ w4[...] + b4[...]
        out[...] = 1.0 / (1.0 + jnp.exp(-l4))

    full = lambda shape: pl.BlockSpec(shape, lambda i: (0, 0))
    return pl.pallas_call(
        body,
        grid=(grid,),
        in_specs=[
            pl.BlockSpec((blk, 128), lambda i: (i, 0)),
            pl.BlockSpec((blk, 128), lambda i: (i + grid, 0)),
            full((128, 512)), full((1, 512)), full((512, 1024)),
            full((1, 1024)), full((1024, 256)), full((1, 256)),
            full((256, 8)), full((1, 8)),
        ],
        out_specs=pl.BlockSpec((blk, 8), lambda i: (i, 0)),
        out_shape=jax.ShapeDtypeStruct((R, 8), _F32),
        compiler_params=pltpu.CompilerParams(
            dimension_semantics=("arbitrary",)),
    )(Zp, Zp, w["W1k"], w["db1_t"], w["W2k"], w["db2_t"], w["W3k"],
      w["db3_t"], w["W4k"], w["db4_t"])


def _fold_weights(p):
    z8 = lambda *s: jnp.zeros(s, _F32)
    bn_s = np.float32(1.0 / np.sqrt(1.0 + 1e-5))
    eye16 = jnp.eye(16, dtype=_F32)
    w = {}
    I8 = jnp.eye(8, dtype=_F32)
    kr = lambda m: jnp.kron(I8, m)
    w_fce = p["fc0_edge_w"][0]
    few_fc = p["fc0_fe_w"]
    u_fc = (jnp.maximum(w_fce, 0) @ few_fc[8:])[None]
    v_fc = (jnp.maximum(-w_fce, 0) @ few_fc[8:])[None]
    w["U8"] = kr(u_fc)
    w["V8"] = kr(v_fc)
    w["K1_fc_k"] = kr(jnp.concatenate([few_fc[:8], z8(8, 16)], 0))
    w["feb_fc_t"] = jnp.tile(p["fc0_fe_b"][None], (1, 8))
    w["ew1_fc_k"] = kr(p["fc1_edge_w"])
    few_sc = p["sc0_fe_w"]
    w["K1_sc_k"] = kr(jnp.concatenate([z8(8, 16), few_sc[:8]], 0))
    w["few_e_sc_k"] = kr(few_sc[8:])
    w["feb_sc_t"] = jnp.tile(p["sc0_fe_b"][None], (1, 8))
    w["ew1_sc_k"] = kr(p["sc1_edge_w"])
    w["S_fc"] = jnp.concatenate([I8, jnp.zeros((24, 8), _F32)], 0)
    w["W_sce_k"] = jnp.concatenate(
        [jnp.zeros((8, 64), _F32)]
        + [kr(p["sc0_edge_w"][c:c + 1]) for c in range(3)], 0)
    w["C0_k"] = kr(eye16[0:1])
    w["C1_k"] = kr(eye16[1:2])
    w["P2_k"] = kr(eye16[2:4])
    w["P4_k"] = kr(eye16[4:12])
    w["P12_k"] = kr(eye16[12:14])
    Wrec_fc = z8(16, 8).at[0].set(jnp.maximum(w_fce, 0)).at[1].set(
        jnp.maximum(-w_fce, 0))
    w["M1_fc"] = Wrec_fc @ p["fc0_fn_w"][:8]
    w["M2_fc"] = jnp.concatenate([p["fc0_fn_w"][8:], z8(8, 16)], 0)
    w["fnb_fc"] = p["fc0_fn_b"][None]
    w["g0_fc"] = (p["fc_bn0_g"] * bn_s)[None]
    w["b0_fc"] = p["fc_bn0_b"][None]
    w["nw1_fc"] = p["fc1_node_w"]
    R24 = z8(16, 2).at[2, 0].set(1.0).at[3, 1].set(1.0)
    w["M3_fc"] = R24 @ p["fc1_fn_w"][:2]
    w["fn1w_b_fc"] = p["fc1_fn_w"][2:]
    w["fn1b_fc"] = p["fc1_fn_b"][None]
    w["g1_fc"] = (p["fc_bn1_g"] * bn_s)[None]
    w["b1_fc"] = p["fc_bn1_b"][None]
    R4_12 = z8(16, 8).at[4:12].set(jnp.eye(8, dtype=_F32))
    w["M1_sc"] = R4_12 @ p["sc0_fn_w"][:8]
    w["M2_sc"] = jnp.concatenate([z8(8, 16), p["sc0_fn_w"][8:]], 0)
    w["fnb_sc"] = p["sc0_fn_b"][None]
    w["g0_sc"] = (p["sc_bn0_g"] * bn_s)[None]
    w["b0_sc"] = p["sc_bn0_b"][None]
    w["nw1_sc"] = p["sc1_node_w"]
    R12_14 = z8(16, 2).at[12, 0].set(1.0).at[13, 1].set(1.0)
    w["M3_sc"] = R12_14 @ p["sc1_fn_w"][:2]
    w["fn1w_b_sc"] = p["sc1_fn_w"][2:]
    w["fn1b_sc"] = p["sc1_fn_b"][None]
    w["g1_sc"] = (p["sc_bn1_g"] * bn_s)[None]
    w["b1_sc"] = p["sc_bn1_b"][None]
    w["Q0"] = jnp.concatenate([jnp.eye(4, dtype=_F32), z8(4, 12)], 1)
    w["Q1"] = jnp.concatenate([z8(4, 4), jnp.eye(4, dtype=_F32), z8(4, 8)], 1)
    w["S0"] = jnp.concatenate([jnp.eye(8, dtype=_F32), z8(8, 8)], 1)
    w["S1"] = jnp.concatenate([z8(8, 8), jnp.eye(8, dtype=_F32)], 1)
    w["W1k"] = kr(jnp.concatenate([p["dec_w1"], z8(8, 64)], 0))
    w["db1_t"] = jnp.tile(p["dec_b1"][None], (1, 8))
    w["W2k"] = kr(p["dec_w2"])
    w["db2_t"] = jnp.tile(p["dec_b2"][None], (1, 8))
    w["W3k"] = kr(p["dec_w3"])
    w["db3_t"] = jnp.tile(p["dec_b3"][None], (1, 8))
    w["W4k"] = kr(p["dec_w4"])
    w["db4_t"] = jnp.tile(p["dec_b4"][None], (1, 8))
    return w


def kernel(fc_x, sc_x, fc_edge_attr, sc_edge_attr, edge_index, params):
    n = fc_x.shape[0]
    w = _fold_weights(params)
    fc_xp = jnp.pad(fc_x, ((0, _NPAD - n), (0, 0)))
    sc_xp = jnp.pad(sc_x, ((0, _NPAD - n), (0, 0)))
    flat_idx = edge_index.reshape(-1)
    zeros_tbl = jnp.zeros((_NPAD, 16), _F32)

    e = edge_index.shape[1]
    hc = _tc_node0(fc_xp, sc_xp, params["fc0_node_w"], params["sc0_node_w"],
                   w["S0"], w["S1"])
    G = _sc_gather16(flat_idx, hc)
    payload_p = _tc_edgeA(G.reshape(2 * e // 8, 128), fc_edge_attr,
                          sc_edge_attr, w, blk=1000)
    partials = _sc_scatter_add16(payload_p.reshape(e, 16), flat_idx, zeros_tbl)
    zpad = _tc_node1(partials, hc, w)
    h = e // 2
    idx_b0 = jnp.concatenate([flat_idx[:h], flat_idx[e:e + h]])
    idx_b1 = jnp.concatenate([flat_idx[h:e], flat_idx[e + h:]])
    Z0 = _sc_gather16(idx_b0, zpad)
    Z1 = _sc_gather16(idx_b1, zpad)
    o0 = _tc_decoder(Z0.reshape(e // 8, 128), w, blk=2000)
    o1 = _tc_decoder(Z1.reshape(e // 8, 128), w, blk=2000)
    return jnp.concatenate([o0, o1], 0).reshape(e, 1)
